# baseline (speedup 1.0000x reference)
import functools

import jax
import jax.numpy as jnp
from jax import lax
from jax.experimental import pallas as pl
from jax.experimental.pallas import tpu as pltpu
from jax.experimental.pallas import tpu_sc as plsc

RATE = 0.1
SCALE = 1.0 / (1.0 - RATE)

B = 4096
L = 50
DIM = 32
NW = 32
BT = B // NW
FT = DIM // 8
NG = BT // 16


def _sc_body(ids_hbm, mask_hbm, w_hbm, out_hbm,
             idst_v, rows_v, rows1_v, mask_v, out_v, sem):
    c = lax.axis_index("c")
    s = lax.axis_index("s")
    wid = s * 2 + c
    b0 = wid * BT

    iota = lax.iota(jnp.int32, 16)

    pltpu.sync_copy(ids_hbm.at[:, pl.ds(b0, BT)], idst_v)

    def l_body(l, carry):
        cp = pltpu.async_copy(w_hbm.at[idst_v.at[l]], rows_v, sem)
        for t in range(FT):
            pltpu.sync_copy(mask_hbm.at[l, t, wid], mask_v.at[t])
        cp.wait()

        def m_body(b, inner):
            for h in range(DIM // 16):
                rows1_v[pl.ds(b * DIM + h * 16, 16)] = rows_v[b, pl.ds(h * 16, 16)]
            return inner

        lax.fori_loop(0, BT, m_body, 0, unroll=8)

        for t in range(FT):
            for f8 in range(8):
                for g in range(NG):
                    sl = pl.ds(g * 16, 16)
                    ix = (iota + g * 16) * DIM + (t * 8 + f8)
                    val = plsc.load_gather(rows1_v, [ix])
                    u = mask_v[t, f8, sl]
                    out_v[t, f8, sl] = val * jnp.where(u >= RATE, SCALE, 0.0)

        for t in range(FT):
            pltpu.sync_copy(out_v.at[t], out_hbm.at[l, t, wid])
        return carry

    lax.fori_loop(0, L, l_body, 0)


@jax.jit
def _run(ids, mask_u, w):
    mask5 = jnp.transpose(
        jnp.transpose(mask_u, (1, 2, 0)).reshape(L, FT, 8, NW, BT),
        (0, 1, 3, 2, 4))
    kern = functools.partial(
        pl.kernel,
        mesh=plsc.VectorSubcoreMesh(core_axis_name="c", subcore_axis_name="s"),
        out_type=jax.ShapeDtypeStruct((L, FT, NW, 8, BT), jnp.float32),
        compiler_params=pltpu.CompilerParams(
            use_tc_tiling_on_sc=False, needs_layout_passes=False),
        scratch_types=[
            pltpu.VMEM((L, BT), jnp.int32),
            pltpu.VMEM((BT, DIM), jnp.float32),
            pltpu.VMEM((BT * DIM,), jnp.float32),
            pltpu.VMEM((FT, 8, BT), jnp.float32),
            pltpu.VMEM((FT, 8, BT), jnp.float32),
            pltpu.SemaphoreType.DMA,
        ],
    )(_sc_body)
    out5 = kern(ids.T, mask5, w)
    return jnp.transpose(
        jnp.transpose(out5, (0, 1, 3, 2, 4)).reshape(L, DIM, B),
        (2, 0, 1))


def kernel(ids, w, mask_u):
    return _run(ids, mask_u, w)


# speedup vs baseline: 1.1196x; 1.1196x over previous
import functools

import jax
import jax.numpy as jnp
from jax import lax
from jax.experimental import pallas as pl
from jax.experimental.pallas import tpu as pltpu
from jax.experimental.pallas import tpu_sc as plsc

RATE = 0.1
SCALE = 1.0 / (1.0 - RATE)

B = 4096
L = 50
DIM = 32
NW = 32
BPW = B // NW
BCHUNK = 16
NCHUNK = BPW // BCHUNK


def _sc_body(ids_hbm, mask_hbm, w_hbm, out_hbm, idx_v, rows_v, mask_v, out_v, sem):
    c = lax.axis_index("c")
    s = lax.axis_index("s")
    wid = s * 2 + c
    b_base = wid * BPW

    def chunk_body(k, carry):
        b0 = b_base + k * BCHUNK
        pltpu.sync_copy(ids_hbm.at[pl.ds(b0, BCHUNK)], idx_v)
        copies = [
            pltpu.async_copy(w_hbm.at[idx_v.at[j]], rows_v.at[j], sem)
            for j in range(BCHUNK)
        ]
        pltpu.sync_copy(mask_hbm.at[pl.ds(b0, BCHUNK)], mask_v)
        for cp in copies:
            cp.wait()

        def row_body(bi, inner):
            def l_body(l, inner2):
                for h in range(DIM // 16):
                    sl = pl.ds(h * 16, 16)
                    u = mask_v[bi, l, sl]
                    scale = jnp.where(u >= RATE, SCALE, 0.0)
                    out_v[bi, l, sl] = rows_v[bi, l, sl] * scale
                return inner2

            return lax.fori_loop(0, L, l_body, inner, unroll=5)

        lax.fori_loop(0, BCHUNK, row_body, 0)

        pltpu.sync_copy(out_v, out_hbm.at[pl.ds(b0, BCHUNK)])
        return carry

    lax.fori_loop(0, NCHUNK, chunk_body, 0)


@jax.jit
def _run(ids, mask_u, w):
    kern = functools.partial(
        pl.kernel,
        mesh=plsc.VectorSubcoreMesh(core_axis_name="c", subcore_axis_name="s"),
        out_type=jax.ShapeDtypeStruct((B, L, DIM), jnp.float32),
        compiler_params=pltpu.CompilerParams(use_tc_tiling_on_sc=False),
        scratch_types=[
            pltpu.VMEM((BCHUNK, L), jnp.int32),
            pltpu.VMEM((BCHUNK, L, DIM), jnp.float32),
            pltpu.VMEM((BCHUNK, L, DIM), jnp.float32),
            pltpu.VMEM((BCHUNK, L, DIM), jnp.float32),
            pltpu.SemaphoreType.DMA,
        ],
    )(_sc_body)
    return kern(ids, mask_u, w)


def kernel(ids, w, mask_u):
    return _run(ids, mask_u, w)
